# baseline (speedup 1.0000x reference)
import functools

import jax
import jax.numpy as jnp
from jax import lax
from jax.experimental import pallas as pl
from jax.experimental.pallas import tpu as pltpu
from jax.experimental.pallas import tpu_sc as plsc

B = 16384
_info = plsc.get_sparse_core_info()
NC, NS, L = _info.num_cores, _info.num_subcores, _info.num_lanes
NW = NC * NS
BPW = B // NW
IDX_CHUNK = 128

_mesh = plsc.VectorSubcoreMesh(core_axis_name="c", subcore_axis_name="s")

_LN2 = 0.6931471805599453


def _log16(x):
    bits = lax.bitcast_convert_type(x, jnp.int32)
    e = (bits >> 23) - 127
    m = lax.bitcast_convert_type((bits & 0x007FFFFF) | 0x3F800000, jnp.float32)
    big = m > 1.4142135381698608
    m = jnp.where(big, m * 0.5, m)
    e = jnp.where(big, e + 1, e)
    r = (m - 1.0) / (m + 1.0)
    r2 = r * r
    logm = r * (2.0 + r2 * (0.6666667 + r2 * (0.4 + r2 * 0.2857143)))
    return e.astype(jnp.float32) * _LN2 + logm


@functools.partial(
    pl.kernel,
    mesh=_mesh,
    out_type=(
        jax.ShapeDtypeStruct((B,), jnp.float32),
        jax.ShapeDtypeStruct((NW * L,), jnp.float32),
    ),
    scratch_types=[
        pltpu.VMEM((BPW,), jnp.int32),
        pltpu.VMEM((BPW,), jnp.int32),
        pltpu.VMEM((BPW,), jnp.float32),
        pltpu.VMEM((BPW,), jnp.float32),
        pltpu.VMEM((BPW,), jnp.float32),
        pltpu.VMEM((BPW,), jnp.float32),
        pltpu.VMEM((BPW,), jnp.float32),
        pltpu.VMEM((L,), jnp.float32),
        pltpu.SemaphoreType.DMA,
        pltpu.SemaphoreType.DMA,
        pltpu.SemaphoreType.DMA,
        pltpu.SemaphoreType.DMA,
        pltpu.SemaphoreType.DMA,
        pltpu.SemaphoreType.DMA,
        pltpu.SemaphoreType.DMA,
        pltpu.SemaphoreType.DMA,
    ],
)
def _sc_fwd(uid_hbm, iid_hbm, sc_hbm, uw_hbm, aw_hbm, bw_hbm,
            out_hbm, parts_hbm,
            uid_v, iid_v, s_v, u_v, a_v, b_v, p_v, acc_v,
            sem_u, sem_i, sem_s, sem_o, *gsems):
    wid = lax.axis_index("s") * NC + lax.axis_index("c")
    base = wid * BPW
    uw1 = uw_hbm.at[0]
    aw1 = aw_hbm.at[0]
    bw1 = bw_hbm.at[0]
    cu = pltpu.async_copy(uid_hbm.at[pl.ds(base, BPW)], uid_v, sem_u)
    ci = pltpu.async_copy(iid_hbm.at[pl.ds(base, BPW)], iid_v, sem_i)
    cs = pltpu.async_copy(sc_hbm.at[pl.ds(base, BPW)], s_v, sem_s)
    nchunks = BPW // IDX_CHUNK
    copies = []
    cu.wait()
    for j in range(nchunks):
        sl = pl.ds(j * IDX_CHUNK, IDX_CHUNK)
        copies.append(pltpu.async_copy(uw1.at[uid_v.at[sl]], u_v.at[sl], gsems[j]))
    ci.wait()
    for j in range(nchunks):
        sl = pl.ds(j * IDX_CHUNK, IDX_CHUNK)
        copies.append(pltpu.async_copy(aw1.at[iid_v.at[sl]], a_v.at[sl], gsems[j]))
        copies.append(pltpu.async_copy(bw1.at[iid_v.at[sl]], b_v.at[sl], gsems[j]))
    cs.wait()
    acc = jnp.zeros((L,), jnp.float32)
    outs = []
    for j in range(nchunks):
        copies[j].wait()
        copies[nchunks + 2 * j].wait()
        copies[nchunks + 2 * j + 1].wait()
        for k in range(IDX_CHUNK // L):
            s16 = pl.ds(j * IDX_CHUNK + k * L, L)
            z = a_v[s16] * (u_v[s16] - b_v[s16])
            q = 1.0 + jnp.exp(-z)
            p_v[s16] = 1.0 / q
            acc = acc + _log16(q) + (1.0 - s_v[s16]) * z
        sl = pl.ds(j * IDX_CHUNK, IDX_CHUNK)
        outs.append(pltpu.async_copy(
            p_v.at[sl], out_hbm.at[pl.ds(base + j * IDX_CHUNK, IDX_CHUNK)], sem_o))
    acc_v[...] = acc
    pltpu.sync_copy(acc_v, parts_hbm.at[pl.ds(wid * L, L)])
    for c in outs:
        c.wait()


def _loss_body(parts_ref, o_ref):
    o_ref[...] = jnp.sum(parts_ref[...]) * (1.0 / B)


_tc_loss = pl.pallas_call(
    _loss_body,
    out_shape=jax.ShapeDtypeStruct((), jnp.float32),
    out_specs=pl.BlockSpec(memory_space=pltpu.SMEM),
)


def kernel(user_id, item_id, score, user_w, a_w, b_w):
    pred, parts = _sc_fwd(user_id.astype(jnp.int32), item_id, score,
                          user_w.T, a_w.T, b_w.T)
    loss = _tc_loss(parts.reshape(4, 128))
    return pred, loss


# speedup vs baseline: 1.0203x; 1.0203x over previous
import functools

import jax
import jax.numpy as jnp
from jax import lax
from jax.experimental import pallas as pl
from jax.experimental.pallas import tpu as pltpu
from jax.experimental.pallas import tpu_sc as plsc

B = 16384
_info = plsc.get_sparse_core_info()
NC, NS, L = _info.num_cores, _info.num_subcores, _info.num_lanes
NW = NC * NS
BPW = B // NW
IDX_CHUNK = 128

_mesh = plsc.VectorSubcoreMesh(core_axis_name="c", subcore_axis_name="s")


@functools.partial(
    pl.kernel,
    mesh=_mesh,
    out_type=jax.ShapeDtypeStruct((B,), jnp.float32),
    scratch_types=[
        pltpu.VMEM((BPW,), jnp.int32),
        pltpu.VMEM((BPW,), jnp.int32),
        pltpu.VMEM((BPW,), jnp.float32),
        pltpu.VMEM((BPW,), jnp.float32),
        pltpu.VMEM((BPW,), jnp.float32),
        pltpu.VMEM((BPW,), jnp.float32),
        pltpu.SemaphoreType.DMA,
        pltpu.SemaphoreType.DMA,
        pltpu.SemaphoreType.DMA,
        pltpu.SemaphoreType.DMA,
        pltpu.SemaphoreType.DMA,
        pltpu.SemaphoreType.DMA,
        pltpu.SemaphoreType.DMA,
    ],
)
def _sc_pred(uid_hbm, iid_hbm, uw_hbm, aw_hbm, bw_hbm, out_hbm,
             uid_v, iid_v, u_v, a_v, b_v, p_v, sem_u, sem_i, sem_o, *gsems):
    wid = lax.axis_index("s") * NC + lax.axis_index("c")
    base = wid * BPW
    uw1 = uw_hbm.at[0]
    aw1 = aw_hbm.at[0]
    bw1 = bw_hbm.at[0]
    cu = pltpu.async_copy(uid_hbm.at[pl.ds(base, BPW)], uid_v, sem_u)
    ci = pltpu.async_copy(iid_hbm.at[pl.ds(base, BPW)], iid_v, sem_i)
    nchunks = BPW // IDX_CHUNK
    copies = []
    cu.wait()
    for j in range(nchunks):
        sl = pl.ds(j * IDX_CHUNK, IDX_CHUNK)
        copies.append(pltpu.async_copy(uw1.at[uid_v.at[sl]], u_v.at[sl], gsems[j]))
    ci.wait()
    for j in range(nchunks):
        sl = pl.ds(j * IDX_CHUNK, IDX_CHUNK)
        copies.append(pltpu.async_copy(aw1.at[iid_v.at[sl]], a_v.at[sl], gsems[j]))
        copies.append(pltpu.async_copy(bw1.at[iid_v.at[sl]], b_v.at[sl], gsems[j]))
    outs = []
    for j in range(nchunks):
        copies[j].wait()
        copies[nchunks + 2 * j].wait()
        copies[nchunks + 2 * j + 1].wait()
        for k in range(IDX_CHUNK // L):
            s16 = pl.ds(j * IDX_CHUNK + k * L, L)
            z = a_v[s16] * (u_v[s16] - b_v[s16])
            p_v[s16] = 1.0 / (1.0 + jnp.exp(-z))
        sl = pl.ds(j * IDX_CHUNK, IDX_CHUNK)
        outs.append(pltpu.async_copy(
            p_v.at[sl], out_hbm.at[pl.ds(base + j * IDX_CHUNK, IDX_CHUNK)], sem_o))
    for c in outs:
        c.wait()


def _loss_body(p_ref, s_ref, o_ref):
    p = p_ref[...]
    s = s_ref[...]
    log_p = jnp.maximum(jnp.log(p), -100.0)
    log_1mp = jnp.maximum(jnp.log(1.0 - p), -100.0)
    o_ref[...] = -jnp.sum(s * log_p + (1.0 - s) * log_1mp) * (1.0 / B)


_tc_loss = pl.pallas_call(
    _loss_body,
    out_shape=jax.ShapeDtypeStruct((), jnp.float32),
    out_specs=pl.BlockSpec(memory_space=pltpu.SMEM),
)


def kernel(user_id, item_id, score, user_w, a_w, b_w):
    pred = _sc_pred(user_id.astype(jnp.int32), item_id,
                    user_w.T, a_w.T, b_w.T)
    loss = _tc_loss(pred.reshape(128, 128), score.reshape(128, 128))
    return pred, loss


# speedup vs baseline: 1.0249x; 1.0046x over previous
import functools

import jax
import jax.numpy as jnp
from jax import lax
from jax.experimental import pallas as pl
from jax.experimental.pallas import tpu as pltpu
from jax.experimental.pallas import tpu_sc as plsc

B = 16384
_info = plsc.get_sparse_core_info()
NC, NS, L = _info.num_cores, _info.num_subcores, _info.num_lanes
NW = NC * NS
BPW = B // NW
IDX_CHUNK = 128

_mesh = plsc.VectorSubcoreMesh(core_axis_name="c", subcore_axis_name="s")


@functools.partial(
    pl.kernel,
    mesh=_mesh,
    out_type=jax.ShapeDtypeStruct((B,), jnp.float32),
    scratch_types=[
        pltpu.VMEM((BPW,), jnp.int32),
        pltpu.VMEM((BPW,), jnp.int32),
        pltpu.VMEM((BPW,), jnp.float32),
        pltpu.VMEM((BPW,), jnp.float32),
        pltpu.VMEM((BPW,), jnp.float32),
        pltpu.VMEM((BPW,), jnp.float32),
        pltpu.SemaphoreType.DMA,
        pltpu.SemaphoreType.DMA,
        pltpu.SemaphoreType.DMA,
        pltpu.SemaphoreType.DMA,
        pltpu.SemaphoreType.DMA,
        pltpu.SemaphoreType.DMA,
        pltpu.SemaphoreType.DMA,
    ],
)
def _sc_pred(uid_hbm, iid_hbm, uw_hbm, aw_hbm, bw_hbm, out_hbm,
             uid_v, iid_v, u_v, a_v, b_v, p_v, sem_u, sem_i, sem_o, *gsems):
    wid = lax.axis_index("s") * NC + lax.axis_index("c")
    base = wid * BPW
    uw1 = uw_hbm.at[0]
    aw1 = aw_hbm.at[0]
    bw1 = bw_hbm.at[0]
    cu = pltpu.async_copy(uid_hbm.at[pl.ds(base, BPW)], uid_v, sem_u)
    ci = pltpu.async_copy(iid_hbm.at[pl.ds(base, BPW)], iid_v, sem_i)
    nchunks = BPW // IDX_CHUNK
    copies = []
    cu.wait()
    for j in range(nchunks):
        sl = pl.ds(j * IDX_CHUNK, IDX_CHUNK)
        copies.append(pltpu.async_copy(uw1.at[uid_v.at[sl]], u_v.at[sl], gsems[j]))
    ci.wait()
    for j in range(nchunks):
        sl = pl.ds(j * IDX_CHUNK, IDX_CHUNK)
        copies.append(pltpu.async_copy(aw1.at[iid_v.at[sl]], a_v.at[sl], gsems[j]))
        copies.append(pltpu.async_copy(bw1.at[iid_v.at[sl]], b_v.at[sl], gsems[j]))
    outs = []
    for j in range(nchunks):
        copies[j].wait()
        copies[nchunks + 2 * j].wait()
        copies[nchunks + 2 * j + 1].wait()
        def _body(k, carry, _j=j):
            s16 = pl.ds(_j * IDX_CHUNK + k * L, L)
            z = a_v[s16] * (u_v[s16] - b_v[s16])
            p_v[s16] = 1.0 / (1.0 + jnp.exp(-z))
            return carry
        lax.fori_loop(0, IDX_CHUNK // L, _body, 0)
        sl = pl.ds(j * IDX_CHUNK, IDX_CHUNK)
        outs.append(pltpu.async_copy(
            p_v.at[sl], out_hbm.at[pl.ds(base + j * IDX_CHUNK, IDX_CHUNK)], sem_o))
    for c in outs:
        c.wait()


def _loss_body(p_ref, s_ref, o_ref):
    p = p_ref[...]
    s = s_ref[...]
    log_p = jnp.maximum(jnp.log(p), -100.0)
    log_1mp = jnp.maximum(jnp.log(1.0 - p), -100.0)
    o_ref[...] = -jnp.sum(s * log_p + (1.0 - s) * log_1mp) * (1.0 / B)


_tc_loss = pl.pallas_call(
    _loss_body,
    out_shape=jax.ShapeDtypeStruct((), jnp.float32),
    out_specs=pl.BlockSpec(memory_space=pltpu.SMEM),
)


def kernel(user_id, item_id, score, user_w, a_w, b_w):
    pred = _sc_pred(user_id.astype(jnp.int32), item_id,
                    user_w.T, a_w.T, b_w.T)
    loss = _tc_loss(pred.reshape(128, 128), score.reshape(128, 128))
    return pred, loss
